# baseline (speedup 1.0000x reference)
import math

import jax
import jax.numpy as jnp
from jax import lax
from jax.experimental import pallas as pl
from jax.experimental.pallas import tpu as pltpu

_M = 0.5
_COS_M = math.cos(_M)
_SIN_M = math.sin(_M)
_TH = math.cos(math.pi - _M)
_MM = math.sin(math.pi - _M) * _M
_EPS = 1e-07

_BR = 8


def _body(s_ref, lab_ref, x_ref, o_ref):
    x = x_ref[...]
    lab = lab_ref[...]
    s = s_ref[0, 0]
    col = lax.broadcasted_iota(jnp.int32, x.shape, 1)
    ct = jnp.clip(x, -1.0 + _EPS, 1.0 - _EPS)
    sine = jnp.sqrt(1.0 - ct * ct)
    phi = ct * _COS_M - sine * _SIN_M
    phi = jnp.where(ct > _TH, phi, ct - _MM)
    o_ref[...] = jnp.where(col == lab, phi, x) * s


def kernel(cosine, labels, s):
    n_rows, n_cols = cosine.shape
    lab2d = labels.astype(jnp.int32).reshape(n_rows, 1)
    s_arr = jnp.asarray(s, jnp.float32).reshape(1, 1)
    grid = (n_rows // _BR,)
    return pl.pallas_call(
        _body,
        grid=grid,
        in_specs=[
            pl.BlockSpec(memory_space=pltpu.SMEM),
            pl.BlockSpec((_BR, 1), lambda i: (i, 0)),
            pl.BlockSpec((_BR, n_cols), lambda i: (i, 0)),
        ],
        out_specs=pl.BlockSpec((_BR, n_cols), lambda i: (i, 0)),
        out_shape=jax.ShapeDtypeStruct((n_rows, n_cols), cosine.dtype),
        compiler_params=pltpu.CompilerParams(
            dimension_semantics=("parallel",),
        ),
    )(s_arr, lab2d, cosine)
